# baseline (speedup 1.0000x reference)
import functools

import jax
import jax.numpy as jnp
from jax import lax
from jax.experimental import pallas as pl
from jax.experimental.pallas import tpu as pltpu
from jax.experimental.pallas import tpu_sc as plsc

VOCAB = 100000
EDIM = 64
B = 4096
L = 200
NUM_TABLES = 4

NC = 2
NS = 16
NW = NC * NS
NSPLIT = 4
BSL = B // NSPLIT
PAIRS = BSL * L
PPW = PAIRS // NW
PCH = 200
NCH = PPW // PCH


def _make_gather():
    mesh = plsc.VectorSubcoreMesh(core_axis_name="c", subcore_axis_name="s",
                                  num_cores=NC)

    @functools.partial(
        pl.kernel,
        mesh=mesh,
        compiler_params=pltpu.CompilerParams(use_tc_tiling_on_sc=False),
        out_type=jax.ShapeDtypeStruct((PAIRS, NUM_TABLES, EDIM),
                                      jnp.float32),
        scratch_types=[
            pltpu.VMEM((NUM_TABLES, PCH), jnp.int32),
            pltpu.VMEM((NUM_TABLES, PCH), jnp.int32),
            pltpu.VMEM((NUM_TABLES, PCH, EDIM), jnp.float32),
            pltpu.VMEM((NUM_TABLES, PCH, EDIM), jnp.float32),
            pltpu.SemaphoreType.DMA,
            pltpu.SemaphoreType.DMA,
            pltpu.SemaphoreType.DMA,
            pltpu.SemaphoreType.DMA,
        ],
    )
    def gather_kernel(t0, t1, t2, t3, i0_hbm, i1_hbm, i2_hbm, i3_hbm,
                      out_hbm, idx_v0, idx_v1, rows_v0, rows_v1,
                      isem, gsem, wsem0, wsem1):
        tables = (t0, t1, t2, t3)
        idxs = (i0_hbm, i1_hbm, i2_hbm, i3_hbm)
        wid = lax.axis_index("s") * NC + lax.axis_index("c")
        w0 = wid * PPW

        def issue_idx(i, dst):
            for t in range(NUM_TABLES):
                pltpu.async_copy(idxs[t].at[pl.ds(w0 + i * PCH, PCH)],
                                 dst.at[t], isem)

        def wait_idx():
            for t in range(NUM_TABLES):
                pltpu.make_async_copy(idxs[t].at[pl.ds(w0, PCH)],
                                      idx_v0.at[t], isem).wait()

        def issue_gather(idx_v, rows_v):
            for t in range(NUM_TABLES):
                pltpu.async_copy(tables[t].at[idx_v.at[t]],
                                 rows_v.at[t], gsem)

        def wait_gather(idx_v, rows_v):
            for t in range(NUM_TABLES):
                pltpu.make_async_copy(tables[t].at[idx_v.at[t]],
                                      rows_v.at[t], gsem).wait()

        def issue_write(i, rows_v, wsem):
            for t in range(NUM_TABLES):
                pltpu.async_copy(rows_v.at[t],
                                 out_hbm.at[pl.ds(w0 + i * PCH, PCH), t],
                                 wsem)

        def wait_write(rows_v, wsem):
            for t in range(NUM_TABLES):
                pltpu.make_async_copy(rows_v.at[t],
                                      out_hbm.at[pl.ds(w0, PCH), t],
                                      wsem).wait()

        issue_idx(0, idx_v0)
        issue_idx(1, idx_v1)
        wait_idx()
        issue_gather(idx_v0, rows_v0)

        def body(k, carry):
            c0 = 2 * k
            c1 = c0 + 1
            wait_gather(idx_v0, rows_v0)
            issue_write(c0, rows_v0, wsem0)

            @pl.when(k > 0)
            def _():
                wait_write(rows_v1, wsem1)

            wait_idx()
            issue_gather(idx_v1, rows_v1)

            @pl.when(c0 + 2 < NCH)
            def _():
                issue_idx(c0 + 2, idx_v0)

            wait_gather(idx_v1, rows_v1)
            issue_write(c1, rows_v1, wsem1)
            wait_write(rows_v0, wsem0)

            @pl.when(c1 + 1 < NCH)
            def _():
                wait_idx()
                issue_gather(idx_v0, rows_v0)

            @pl.when(c1 + 2 < NCH)
            def _():
                issue_idx(c1 + 2, idx_v1)

            return carry

        lax.fori_loop(0, NCH // 2, body, 0)
        wait_write(rows_v1, wsem1)

    return gather_kernel


_gather = _make_gather()


def kernel(num_mentions_total, num_mentions_named, num_mentions_nominal,
           num_mentions_pronominal, W_total, W_named, W_nominal,
           W_pronominal):
    idx_flat = [m.astype(jnp.int32).reshape(-1)
                for m in (num_mentions_total, num_mentions_named,
                          num_mentions_nominal, num_mentions_pronominal)]
    slices = []
    for k in range(NSPLIT):
        out_k = _gather(W_total, W_named, W_nominal, W_pronominal,
                        *(ix[k * PAIRS:(k + 1) * PAIRS] for ix in idx_flat))
        slices.append(out_k.reshape(BSL, L, NUM_TABLES * EDIM))
    return jnp.concatenate(slices, axis=0)


# speedup vs baseline: 3.9089x; 3.9089x over previous
import functools

import jax
import jax.numpy as jnp
from jax import lax
from jax.experimental import pallas as pl
from jax.experimental.pallas import tpu as pltpu
from jax.experimental.pallas import tpu_sc as plsc

VOCAB = 100000
EDIM = 64
B = 4096
L = 200
NUM_TABLES = 4

NC = 2
NS = 16
NW = NC * NS
PAIRS = B * L
PPW = PAIRS // NW
PCH = L
NCH = PPW // PCH
NG = PCH // 8


def _make_gather():
    mesh = plsc.VectorSubcoreMesh(core_axis_name="c", subcore_axis_name="s",
                                  num_cores=NC)

    @functools.partial(
        pl.kernel,
        mesh=mesh,
        compiler_params=pltpu.CompilerParams(use_tc_tiling_on_sc=False),
        out_type=jax.ShapeDtypeStruct((PAIRS // 8, 2, 8, 128), jnp.float32),
        scratch_types=[
            pltpu.VMEM((NUM_TABLES, PCH), jnp.int32),
            pltpu.VMEM((NUM_TABLES, PCH), jnp.int32),
            pltpu.VMEM((NUM_TABLES, PCH, EDIM), jnp.float32),
            pltpu.VMEM((NUM_TABLES, PCH, EDIM), jnp.float32),
            pltpu.SemaphoreType.DMA,
            pltpu.SemaphoreType.DMA,
            pltpu.SemaphoreType.DMA,
            pltpu.SemaphoreType.DMA,
        ],
    )
    def gather_kernel(t0, t1, t2, t3, i0_hbm, i1_hbm, i2_hbm, i3_hbm,
                      out_hbm, idx_v0, idx_v1, rows_v0, rows_v1,
                      isem, gsem, wsem0, wsem1):
        tables = (t0, t1, t2, t3)
        idxs = (i0_hbm, i1_hbm, i2_hbm, i3_hbm)
        wid = lax.axis_index("s") * NC + lax.axis_index("c")
        w0 = wid * PPW

        def issue_idx(i, dst):
            for t in range(NUM_TABLES):
                pltpu.async_copy(idxs[t].at[pl.ds(w0 + i * PCH, PCH)],
                                 dst.at[t], isem)

        def wait_idx():
            for t in range(NUM_TABLES):
                pltpu.make_async_copy(idxs[t].at[pl.ds(w0, PCH)],
                                      idx_v0.at[t], isem).wait()

        def issue_gather(idx_v, rows_v):
            for t in range(NUM_TABLES):
                pltpu.async_copy(tables[t].at[idx_v.at[t]],
                                 rows_v.at[t], gsem)

        def wait_gather(idx_v, rows_v):
            for t in range(NUM_TABLES):
                pltpu.make_async_copy(tables[t].at[idx_v.at[t]],
                                      rows_v.at[t], gsem).wait()

        def issue_write(i, rows_v, wsem):
            g0 = (w0 + i * PCH) // 8

            def wbody(g, carry):
                for t in range(NUM_TABLES):
                    pltpu.async_copy(
                        rows_v.at[t, pl.ds(8 * g, 8)],
                        out_hbm.at[g0 + g, t // 2, :,
                                   pl.ds((t % 2) * EDIM, EDIM)],
                        wsem)
                return carry

            lax.fori_loop(0, NG, wbody, 0)

        def wait_write(rows_v, wsem):
            def wbody(g, carry):
                for t in range(NUM_TABLES):
                    pltpu.make_async_copy(
                        rows_v.at[t, pl.ds(0, 8)],
                        out_hbm.at[w0 // 8, t // 2, :,
                                   pl.ds((t % 2) * EDIM, EDIM)],
                        wsem).wait()
                return carry

            lax.fori_loop(0, NG, wbody, 0)

        issue_idx(0, idx_v0)
        issue_idx(1, idx_v1)
        wait_idx()
        issue_gather(idx_v0, rows_v0)

        def body(k, carry):
            c0 = 2 * k
            c1 = c0 + 1
            wait_gather(idx_v0, rows_v0)
            issue_write(c0, rows_v0, wsem0)

            @pl.when(k > 0)
            def _():
                wait_write(rows_v1, wsem1)

            wait_idx()
            issue_gather(idx_v1, rows_v1)

            @pl.when(c0 + 2 < NCH)
            def _():
                issue_idx(c0 + 2, idx_v0)

            wait_gather(idx_v1, rows_v1)
            issue_write(c1, rows_v1, wsem1)
            wait_write(rows_v0, wsem0)

            @pl.when(c1 + 1 < NCH)
            def _():
                wait_idx()
                issue_gather(idx_v0, rows_v0)

            @pl.when(c1 + 2 < NCH)
            def _():
                issue_idx(c1 + 2, idx_v1)

            return carry

        lax.fori_loop(0, NCH // 2, body, 0)
        wait_write(rows_v1, wsem1)

    return gather_kernel


_gather = _make_gather()


def kernel(num_mentions_total, num_mentions_named, num_mentions_nominal,
           num_mentions_pronominal, W_total, W_named, W_nominal,
           W_pronominal):
    out4 = _gather(W_total, W_named, W_nominal, W_pronominal,
                   num_mentions_total.astype(jnp.int32).reshape(-1),
                   num_mentions_named.astype(jnp.int32).reshape(-1),
                   num_mentions_nominal.astype(jnp.int32).reshape(-1),
                   num_mentions_pronominal.astype(jnp.int32).reshape(-1))
    out5 = out4.reshape(B, L // 8, 2, 8, 128).transpose(0, 1, 3, 2, 4)
    return out5.reshape(B, L, NUM_TABLES * EDIM)
